# baseline (speedup 1.0000x reference)
import jax
import jax.numpy as jnp
from jax import lax
from jax.experimental import pallas as pl
from jax.experimental.pallas import tpu as pltpu

_CB_DIM = 4
_MAXL = 8
_BASIS = (1, 8, 64, 256)
_RING = 4


def _compute(z, w_in_ref, b_in_ref, w_out_ref, b_out_ref, v_ref):
    zp = jnp.dot(w_in_ref[...], z,
                 preferred_element_type=jnp.float32) + b_in_ref[...]

    best = jnp.full(zp.shape, jnp.inf, jnp.float32)
    q = jnp.zeros(zp.shape, jnp.float32)
    kidx = jnp.zeros(zp.shape, jnp.int32)
    for k in range(_MAXL):
        vk = v_ref[:, k:k + 1]
        dist = jnp.abs(zp - vk)
        better = dist < best
        best = jnp.where(better, dist, best)
        q = jnp.where(better, jnp.broadcast_to(vk, zp.shape), q)
        kidx = jnp.where(better, k, kidx)

    idx = (kidx[0:1] * _BASIS[0] + kidx[1:2] * _BASIS[1]
           + kidx[2:3] * _BASIS[2] + kidx[3:4] * _BASIS[3])
    out = jnp.dot(w_out_ref[...], q,
                  preferred_element_type=jnp.float32) + b_out_ref[...]
    part = jnp.sum((z - out) * (z - out))
    return out, idx, part


def _body(z_hbm, w_in_ref, b_in_ref, w_out_ref, b_out_ref, v_ref,
          out_hbm, idx_ref, loss_ref, *scratch):
    ibufs = scratch[:_RING]
    obufs = scratch[_RING:2 * _RING]
    sin = scratch[2 * _RING:3 * _RING]
    sout = scratch[3 * _RING:4 * _RING]
    nb = z_hbm.shape[0]

    for s in range(_RING):
        pltpu.make_async_copy(z_hbm.at[pl.ds(s, 1)], ibufs[s], sin[s]).start()

    def step(i, acc):
        res = acc
        for s in range(_RING):
            @pl.when(lax.rem(i, _RING) == s)
            def _():
                pltpu.make_async_copy(z_hbm.at[pl.ds(i, 1)], ibufs[s], sin[s]).wait()

                out, idx, part = _compute(
                    ibufs[s][0], w_in_ref, b_in_ref, w_out_ref, b_out_ref,
                    v_ref)
                idx_ref[i] = idx

                @pl.when(i >= _RING)
                def _():
                    pltpu.make_async_copy(
                        obufs[s], out_hbm.at[pl.ds(i, 1)], sout[s]).wait()

                obufs[s][0] = out
                pltpu.make_async_copy(obufs[s], out_hbm.at[pl.ds(i, 1)], sout[s]).start()

                @pl.when(i + _RING < nb)
                def _():
                    pltpu.make_async_copy(
                        z_hbm.at[pl.ds(i + _RING, 1)], ibufs[s], sin[s]).start()

                loss_ref[0, 0] += part
        return res

    loss_ref[0, 0] = 0.0
    lax.fori_loop(0, nb, step, 0)
    for s in range(_RING):
        pltpu.make_async_copy(obufs[s], out_hbm.at[pl.ds(0, 1)], sout[s]).wait()


def kernel(z, W_in, b_in, W_out, b_out, v0, v1, v2, v3):
    b, d, h, w = z.shape
    n = h * w
    zf = z.reshape(b, d, n)

    vmat = jnp.full((_CB_DIM, _MAXL), 1e30, jnp.float32)
    vmat = vmat.at[0, :v0.shape[0]].set(v0)
    vmat = vmat.at[1, :v1.shape[0]].set(v1)
    vmat = vmat.at[2, :v2.shape[0]].set(v2)
    vmat = vmat.at[3, :v3.shape[0]].set(v3)

    out, idx, loss_sum = pl.pallas_call(
        _body,
        in_specs=[
            pl.BlockSpec(memory_space=pltpu.MemorySpace.HBM),
            pl.BlockSpec((_CB_DIM, d), lambda: (0, 0)),
            pl.BlockSpec((_CB_DIM, 1), lambda: (0, 0)),
            pl.BlockSpec((d, _CB_DIM), lambda: (0, 0)),
            pl.BlockSpec((d, 1), lambda: (0, 0)),
            pl.BlockSpec((_CB_DIM, _MAXL), lambda: (0, 0)),
        ],
        out_specs=[
            pl.BlockSpec(memory_space=pltpu.MemorySpace.HBM),
            pl.BlockSpec((b, 1, n), lambda: (0, 0, 0)),
            pl.BlockSpec((1, 1), lambda: (0, 0), memory_space=pltpu.SMEM),
        ],
        out_shape=[
            jax.ShapeDtypeStruct((b, d, n), jnp.float32),
            jax.ShapeDtypeStruct((b, 1, n), jnp.int32),
            jax.ShapeDtypeStruct((1, 1), jnp.float32),
        ],
        scratch_shapes=(
            [pltpu.VMEM((1, d, n), jnp.float32)] * (2 * _RING)
            + [pltpu.SemaphoreType.DMA] * (2 * _RING)
        ),
    )(zf, W_in, b_in.reshape(_CB_DIM, 1), W_out, b_out.reshape(d, 1), vmat)

    out = out.reshape(b, d, h, w)
    indices = idx.reshape(b, h, w)
    loss = 0.2 * loss_sum[0, 0] / (b * d * n)
    return out, indices, loss


# speedup vs baseline: 1.0021x; 1.0021x over previous
import jax
import jax.numpy as jnp
from jax.experimental import pallas as pl
from jax.experimental.pallas import tpu as pltpu

_CB_DIM = 4
_MAXL = 8
_BASIS = (1, 8, 64, 256)
_BB = 4


def _body(z_ref, w_in_ref, b_in_ref, w_out_ref, b_out_ref, v_ref,
          out_ref, idx_ref, loss_ref):
    i = pl.program_id(0)
    part = jnp.float32(0.0)
    for bb in range(_BB):
        z = z_ref[bb]
        zp = jnp.dot(w_in_ref[...], z,
                     preferred_element_type=jnp.float32) + b_in_ref[...]

        best = jnp.full(zp.shape, jnp.inf, jnp.float32)
        q = jnp.zeros(zp.shape, jnp.float32)
        kidx = jnp.zeros(zp.shape, jnp.int32)
        for k in range(_MAXL):
            vk = v_ref[:, k:k + 1]
            dist = jnp.abs(zp - vk)
            better = dist < best
            best = jnp.where(better, dist, best)
            q = jnp.where(better, jnp.broadcast_to(vk, zp.shape), q)
            kidx = jnp.where(better, k, kidx)

        idx_ref[bb] = (kidx[0:1] * _BASIS[0] + kidx[1:2] * _BASIS[1]
                       + kidx[2:3] * _BASIS[2] + kidx[3:4] * _BASIS[3])

        out = jnp.dot(w_out_ref[...], q,
                      preferred_element_type=jnp.float32) + b_out_ref[...]
        out_ref[bb] = out

        diff = z - out
        part = part + jnp.sum(diff * diff)

    @pl.when(i == 0)
    def _init():
        loss_ref[0, 0] = part

    @pl.when(i > 0)
    def _acc():
        loss_ref[0, 0] += part


def kernel(z, W_in, b_in, W_out, b_out, v0, v1, v2, v3):
    b, d, h, w = z.shape
    n = h * w
    zf = z.reshape(b, d, n)

    vmat = jnp.full((_CB_DIM, _MAXL), 1e30, jnp.float32)
    vmat = vmat.at[0, :v0.shape[0]].set(v0)
    vmat = vmat.at[1, :v1.shape[0]].set(v1)
    vmat = vmat.at[2, :v2.shape[0]].set(v2)
    vmat = vmat.at[3, :v3.shape[0]].set(v3)

    out, idx, loss_sum = pl.pallas_call(
        _body,
        grid=(b // _BB,),
        in_specs=[
            pl.BlockSpec((_BB, d, n), lambda i: (i, 0, 0)),
            pl.BlockSpec((_CB_DIM, d), lambda i: (0, 0)),
            pl.BlockSpec((_CB_DIM, 1), lambda i: (0, 0)),
            pl.BlockSpec((d, _CB_DIM), lambda i: (0, 0)),
            pl.BlockSpec((d, 1), lambda i: (0, 0)),
            pl.BlockSpec((_CB_DIM, _MAXL), lambda i: (0, 0)),
        ],
        out_specs=[
            pl.BlockSpec((_BB, d, n), lambda i: (i, 0, 0)),
            pl.BlockSpec((_BB, 1, n), lambda i: (i, 0, 0)),
            pl.BlockSpec((1, 1), lambda i: (0, 0), memory_space=pltpu.SMEM),
        ],
        out_shape=[
            jax.ShapeDtypeStruct((b, d, n), jnp.float32),
            jax.ShapeDtypeStruct((b, 1, n), jnp.int32),
            jax.ShapeDtypeStruct((1, 1), jnp.float32),
        ],
    )(zf, W_in, b_in.reshape(_CB_DIM, 1), W_out, b_out.reshape(d, 1), vmat)

    out = out.reshape(b, d, h, w)
    indices = idx.reshape(b, h, w)
    loss = 0.2 * loss_sum[0, 0] / (b * d * n)
    return out, indices, loss
